# Initial kernel scaffold; baseline (speedup 1.0000x reference)
import functools

import jax
import jax.numpy as jnp
from jax import lax
from jax.experimental import pallas as pl
from jax.experimental.pallas import tpu as pltpu
from jax.experimental.pallas import tpu_sc as plsc

N = 10000
E = 320000
D = 128

NC = 2
NS = 16
G = 128
NGROUPS = E // G
ROWS_PER_TILE = N // NS


def _linear_body(x_ref, wt_ref, b_ref, o_ref):
    o_ref[...] = (
        jnp.dot(x_ref[...], wt_ref[...], preferred_element_type=jnp.float32)
        + b_ref[...]
    )


def _linear(x, wt, b2d):
    blk = 2000
    return pl.pallas_call(
        _linear_body,
        grid=(N // blk,),
        in_specs=[
            pl.BlockSpec((blk, D), lambda i: (i, 0)),
            pl.BlockSpec((D, D), lambda i: (0, 0)),
            pl.BlockSpec((1, D), lambda i: (0, 0)),
        ],
        out_specs=pl.BlockSpec((blk, D), lambda i: (i, 0)),
        out_shape=jax.ShapeDtypeStruct((N, D), jnp.float32),
    )(x, wt, b2d)


def _combine_body(p_ref, o_ref):
    o_ref[...] = jnp.maximum(p_ref[0] + p_ref[1], 0.0)


def _combine(partials):
    blk = 2000
    return pl.pallas_call(
        _combine_body,
        grid=(N // blk,),
        in_specs=[pl.BlockSpec((NC, blk, D), lambda i: (0, i, 0))],
        out_specs=pl.BlockSpec((blk, D), lambda i: (i, 0)),
        out_shape=jax.ShapeDtypeStruct((N, D), jnp.float32),
    )(partials)


def _spmm_body(h_hbm, src_hbm, dst_hbm, w_hbm, zeros_hbm, out_hbm,
               idx_src, idx_dst, wts, rows, acc, sem):
    c = lax.axis_index("c")
    s = lax.axis_index("s")
    wid = c * NS + s

    row0 = s * ROWS_PER_TILE
    pltpu.sync_copy(zeros_hbm.at[pl.ds(row0, ROWS_PER_TILE)],
                    acc.at[pl.ds(row0, ROWS_PER_TILE)])
    plsc.subcore_barrier()

    @pl.loop(wid, NGROUPS, step=NC * NS)
    def _group(j):
        base = j * G
        pltpu.sync_copy(src_hbm.at[pl.ds(base, G)], idx_src)
        pltpu.sync_copy(dst_hbm.at[pl.ds(base, G)], idx_dst)
        pltpu.sync_copy(w_hbm.at[pl.ds(base, G)], wts)
        pltpu.async_copy(h_hbm.at[idx_src], rows, sem).wait()

        @pl.loop(0, G)
        def _edge(e):
            wv = wts[e]
            for jj in range(D // 16):
                sl = pl.ds(jj * 16, 16)
                rows[e, sl] = rows[e, sl] * wv

        pltpu.sync_copy(rows, acc.at[idx_dst], add=True)

    plsc.subcore_barrier()
    pltpu.sync_copy(acc.at[pl.ds(row0, ROWS_PER_TILE)],
                    out_hbm.at[c, pl.ds(row0, ROWS_PER_TILE)])


def _spmm(h, src, dst, w, zeros):
    mesh = plsc.VectorSubcoreMesh(core_axis_name="c", subcore_axis_name="s")
    kern = pl.kernel(
        _spmm_body,
        out_type=jax.ShapeDtypeStruct((NC, N, D), jnp.float32),
        mesh=mesh,
        scratch_types=[
            pltpu.VMEM((G,), jnp.int32),
            pltpu.VMEM((G,), jnp.int32),
            pltpu.VMEM((G,), jnp.float32),
            pltpu.VMEM((G, D), jnp.float32),
            pltpu.VMEM_SHARED((N, D), jnp.float32),
            pltpu.SemaphoreType.DMA,
        ],
    )
    return kern(h, src, dst, w, zeros)


def kernel(x, edge_index, edge_weight, W, b):
    wt = W.T
    b2d = b.reshape(1, D)
    h = _linear(x, wt, b2d)
    src = edge_index[1]
    dst = edge_index[0]
    zeros = jnp.zeros((N, D), dtype=jnp.float32)
    partials = _spmm(h, src, dst, edge_weight, zeros)
    return _combine(partials)


# speedup vs baseline: 5.3701x; 5.3701x over previous
import functools

import jax
import jax.numpy as jnp
from jax import lax
from jax.experimental import pallas as pl
from jax.experimental.pallas import tpu as pltpu
from jax.experimental.pallas import tpu_sc as plsc

N = 10000
E = 320000
D = 128

NC = 2
NS = 16
G = 128
NGROUPS = E // G
ROWS_A = 632
ROWS_B = N - (NS - 1) * ROWS_A


def _linear_body(x_ref, wt_ref, b_ref, o_ref):
    o_ref[...] = (
        jnp.dot(x_ref[...], wt_ref[...], preferred_element_type=jnp.float32)
        + b_ref[...]
    )


def _linear(x, wt, b2d):
    blk = 2000
    return pl.pallas_call(
        _linear_body,
        grid=(N // blk,),
        in_specs=[
            pl.BlockSpec((blk, D), lambda i: (i, 0)),
            pl.BlockSpec((D, D), lambda i: (0, 0)),
            pl.BlockSpec((1, D), lambda i: (0, 0)),
        ],
        out_specs=pl.BlockSpec((blk, D), lambda i: (i, 0)),
        out_shape=jax.ShapeDtypeStruct((N, D), jnp.float32),
    )(x, wt, b2d)


def _combine_body(p_ref, o_ref):
    o_ref[...] = jnp.maximum(p_ref[0] + p_ref[1], 0.0)


def _combine(partials):
    blk = 2000
    return pl.pallas_call(
        _combine_body,
        grid=(N // blk,),
        in_specs=[pl.BlockSpec((NC, blk, D), lambda i: (0, i, 0))],
        out_specs=pl.BlockSpec((blk, D), lambda i: (i, 0)),
        out_shape=jax.ShapeDtypeStruct((N, D), jnp.float32),
    )(partials)


def _spmm_body(h_hbm, src_hbm, dst_hbm, w_hbm, zeros_hbm, out_hbm,
               idx_src, idx_dst, wts, rows, acc, sem):
    c = lax.axis_index("c")
    s = lax.axis_index("s")
    wid = c * NS + s

    row0 = s * ROWS_A

    @pl.when(s < NS - 1)
    def _():
        pltpu.sync_copy(zeros_hbm.at[pl.ds(row0, ROWS_A)],
                        acc.at[pl.ds(row0, ROWS_A)])

    @pl.when(s == NS - 1)
    def _():
        pltpu.sync_copy(zeros_hbm.at[pl.ds(row0, ROWS_B)],
                        acc.at[pl.ds(row0, ROWS_B)])

    plsc.subcore_barrier()

    @pl.loop(wid, NGROUPS, step=NC * NS)
    def _group(j):
        base = j * G
        pltpu.sync_copy(src_hbm.at[pl.ds(base, G)], idx_src)
        pltpu.sync_copy(dst_hbm.at[pl.ds(base, G)], idx_dst)
        pltpu.sync_copy(w_hbm.at[pl.ds(base, G)], wts)
        pltpu.async_copy(h_hbm.at[idx_src], rows, sem).wait()

        @pl.loop(0, G // 16)
        def _edge16(g):
            wv16 = wts[pl.ds(g * 16, 16)]
            for i in range(16):
                w = wv16[i]
                e = g * 16 + i
                for jj in range(D // 16):
                    sl = pl.ds(jj * 16, 16)
                    rows[e, sl] = rows[e, sl] * w

        pltpu.sync_copy(rows, acc.at[idx_dst], add=True)

    plsc.subcore_barrier()

    @pl.when(s < NS - 1)
    def _():
        pltpu.sync_copy(acc.at[pl.ds(row0, ROWS_A)],
                        out_hbm.at[c, pl.ds(row0, ROWS_A)])

    @pl.when(s == NS - 1)
    def _():
        pltpu.sync_copy(acc.at[pl.ds(row0, ROWS_B)],
                        out_hbm.at[c, pl.ds(row0, ROWS_B)])


def _spmm(h, src, dst, w, zeros):
    mesh = plsc.VectorSubcoreMesh(core_axis_name="c", subcore_axis_name="s")
    kern = pl.kernel(
        _spmm_body,
        out_type=jax.ShapeDtypeStruct((NC, N, D), jnp.float32),
        mesh=mesh,
        scratch_types=[
            pltpu.VMEM((G,), jnp.int32),
            pltpu.VMEM((G,), jnp.int32),
            pltpu.VMEM((G,), jnp.float32),
            pltpu.VMEM((G, D), jnp.float32),
            pltpu.VMEM_SHARED((N, D), jnp.float32),
            pltpu.SemaphoreType.DMA,
        ],
    )
    return kern(h, src, dst, w, zeros)


def kernel(x, edge_index, edge_weight, W, b):
    wt = W.T
    b2d = b.reshape(1, D)
    h = _linear(x, wt, b2d)
    src = edge_index[1]
    dst = edge_index[0]
    zeros = jnp.zeros((N, D), dtype=jnp.float32)
    partials = _spmm(h, src, dst, edge_weight, zeros)
    return _combine(partials)
